# baseline (speedup 1.0000x reference)
import functools

import jax
import jax.numpy as jnp
from jax import lax
from jax.experimental import pallas as pl
from jax.experimental.pallas import tpu as pltpu
from jax.experimental.pallas import tpu_sc as plsc

EMB = 16
NC = 2
NS = 16
BL = 128
KJ = 16
VP = 1 << 20
H = VP // NC
VS = H // NS


@functools.lru_cache(maxsize=None)
def _make_sc_hist(n_rows: int):
    chunk_rows = n_rows // NS
    assert chunk_rows * NS == n_rows
    R = chunk_rows // KJ
    assert R * KJ == chunk_rows and R % 2 == 0
    NK = R // 2

    mesh = plsc.VectorSubcoreMesh(
        core_axis_name="c", subcore_axis_name="s",
        num_cores=NC, num_subcores=NS)

    @functools.partial(
        pl.kernel,
        out_type=jax.ShapeDtypeStruct((NC, H), jnp.float32),
        mesh=mesh,
        scratch_types=[
            pltpu.VMEM((KJ, BL), jnp.int32),
            pltpu.VMEM((KJ, BL), jnp.int32),
            pltpu.VMEM((KJ, BL), jnp.int32),
            pltpu.VMEM((KJ, BL), jnp.int32),
            pltpu.VMEM((BL,), jnp.float32),
            pltpu.VMEM((VS,), jnp.float32),
            pltpu.VMEM_SHARED((H + BL,), jnp.float32),
            pltpu.SemaphoreType.DMA,
            pltpu.SemaphoreType.DMA,
            pltpu.SemaphoreType.DMA,
            pltpu.SemaphoreType.DMA,
        ],
    )
    def sc_hist(idx_hbm, out_hbm,
                idx0, idx1, rmp0, rmp1, ones_v, stage_v, hist_sh,
                si0, si1, ss0, ss1):
        cid = lax.axis_index("c")
        sid = lax.axis_index("s")
        base = sid * chunk_rows
        lo = cid * H

        zero16 = jnp.zeros((16,), jnp.float32)

        def zbody(i, carry):
            stage_v[pl.ds(i * 16, 16)] = zero16
            return carry

        lax.fori_loop(0, VS // 16, zbody, 0)
        one = jnp.ones((16,), jnp.float32)
        for j in range(BL // 16):
            ones_v[pl.ds(j * 16, 16)] = one
        pltpu.sync_copy(stage_v, hist_sh.at[pl.ds(sid * VS, VS)])
        plsc.subcore_barrier()

        def idx_copy(r, buf, sem):
            return pltpu.make_async_copy(
                idx_hbm.at[pl.ds(base + r * KJ, KJ)], buf, sem)

        hcap = jnp.uint32(H)

        def remap(idxbuf, rmpbuf):
            def body(i, carry):
                v = idxbuf[i // (BL // 16), pl.ds((i % (BL // 16)) * 16, 16)]
                t = v - lo
                tu = lax.bitcast_convert_type(t, jnp.uint32)
                c = jnp.minimum(tu, hcap)
                rmpbuf[i // (BL // 16), pl.ds((i % (BL // 16)) * 16, 16)] = (
                    lax.bitcast_convert_type(c, jnp.int32))
                return carry
            return lax.fori_loop(0, KJ * (BL // 16), body, 0)

        def start_scatters(rmpbuf, sem):
            for j in range(KJ):
                pltpu.async_copy(ones_v, hist_sh.at[rmpbuf.at[j]], sem,
                                 add=True)

        def drain_scatters(rmpbuf, sem):
            for j in range(KJ):
                pltpu.make_async_copy(
                    ones_v, hist_sh.at[rmpbuf.at[j]], sem).wait()

        idx_copy(0, idx0, si0).start()
        idx_copy(1, idx1, si1).start()
        idx_copy(0, idx0, si0).wait()
        remap(idx0, rmp0)

        def round_pair(k, carry):
            start_scatters(rmp0, ss0)

            @pl.when(k + 1 < NK)
            def _():
                idx_copy(2 * k + 2, idx0, si0).start()

            idx_copy(2 * k + 1, idx1, si1).wait()
            remap(idx1, rmp1)
            start_scatters(rmp1, ss1)
            drain_scatters(rmp0, ss0)

            @pl.when(k + 1 < NK)
            def _():
                idx_copy(2 * k + 2, idx0, si0).wait()
                remap(idx0, rmp0)
                idx_copy(2 * k + 3, idx1, si1).start()

            drain_scatters(rmp1, ss1)
            return carry

        lax.fori_loop(0, NK, round_pair, 0)
        plsc.subcore_barrier()

        pltpu.sync_copy(hist_sh.at[pl.ds(sid * VS, VS)], stage_v)
        pltpu.sync_copy(stage_v, out_hbm.at[cid, pl.ds(sid * VS, VS)])

    return sc_hist


def _tc_matvec(h8, embpT, W, b2, inv_l):
    C = VP // 8

    def body(h_ref, e_ref, w_ref, b_ref, o_ref, acc_ref):
        i = pl.program_id(0)

        @pl.when(i == 0)
        def _():
            acc_ref[...] = jnp.zeros_like(acc_ref)

        acc_ref[...] += lax.dot_general(
            h_ref[0], e_ref[...], (((1,), (1,)), ((), ())),
            preferred_element_type=jnp.float32)

        @pl.when(i == pl.num_programs(0) - 1)
        def _():
            o_ref[...] = lax.dot_general(
                acc_ref[...] * inv_l, w_ref[...],
                (((1,), (1,)), ((), ())),
                preferred_element_type=jnp.float32) + b_ref[...]

    return pl.pallas_call(
        body,
        grid=(8,),
        in_specs=[
            pl.BlockSpec((1, 1, C), lambda i: (i, 0, 0)),
            pl.BlockSpec((EMB, C), lambda i: (0, i)),
            pl.BlockSpec(W.shape, lambda i: (0, 0)),
            pl.BlockSpec(b2.shape, lambda i: (0, 0)),
        ],
        out_specs=pl.BlockSpec((1, b2.shape[1]), lambda i: (0, 0)),
        out_shape=jax.ShapeDtypeStruct((1, b2.shape[1]), jnp.float32),
        scratch_shapes=[pltpu.VMEM((1, EMB), jnp.float32)],
    )(h8, embpT, W, b2)


def kernel(inputs, emb, W, b):
    L = inputs.shape[0]
    V = emb.shape[0]
    idx2d = inputs.astype(jnp.int32).reshape(L // BL, BL)
    hist2 = _make_sc_hist(L // BL)(idx2d)
    h8 = hist2.reshape(8, 1, VP // 8)
    embpT = jnp.pad(emb, ((0, VP - V), (0, 0))).T
    return _tc_matvec(h8, embpT, W, b.reshape(1, -1), 1.0 / L)


# speedup vs baseline: 2.6343x; 2.6343x over previous
import functools

import jax
import jax.numpy as jnp
from jax import lax
from jax.experimental import pallas as pl
from jax.experimental.pallas import tpu as pltpu
from jax.experimental.pallas import tpu_sc as plsc

EMB = 16
NC = 2
NS = 16
NW = NC * NS
BL = 128
KJ = 16
B = KJ * BL
VP = 1 << 20
TC_C = 8192


@functools.lru_cache(maxsize=None)
def _make_sc_sum(n_rows: int):
    chunk_rows = n_rows // NW
    assert chunk_rows * NW == n_rows
    R = chunk_rows // KJ
    assert R * KJ == chunk_rows and R % 2 == 0
    NK = R // 2

    mesh = plsc.VectorSubcoreMesh(
        core_axis_name="c", subcore_axis_name="s",
        num_cores=NC, num_subcores=NS)

    @functools.partial(
        pl.kernel,
        out_type=jax.ShapeDtypeStruct((NW, EMB), jnp.float32),
        mesh=mesh,
        compiler_params=pltpu.CompilerParams(use_tc_tiling_on_sc=False),
        scratch_types=[
            pltpu.VMEM((KJ, BL), jnp.int32),
            pltpu.VMEM((KJ, BL), jnp.int32),
            pltpu.VMEM((B, EMB), jnp.float32),
            pltpu.VMEM((B, EMB), jnp.float32),
            pltpu.VMEM((EMB,), jnp.float32),
            pltpu.SemaphoreType.DMA,
            pltpu.SemaphoreType.DMA,
            pltpu.SemaphoreType.DMA,
            pltpu.SemaphoreType.DMA,
        ],
    )
    def sc_sum(idx_hbm, emb_hbm, out_hbm,
               idx0, idx1, rows0, rows1, accv, si0, si1, sg0, sg1):
        wid = lax.axis_index("s") * NC + lax.axis_index("c")
        base = wid * chunk_rows

        def idx_copy(r, buf, sem):
            return pltpu.make_async_copy(
                idx_hbm.at[pl.ds(base + r * KJ, KJ)], buf, sem)

        def start_gathers(idxbuf, rowbuf, sem):
            for j in range(KJ):
                pltpu.make_async_copy(
                    emb_hbm.at[idxbuf.at[j]],
                    rowbuf.at[pl.ds(j * BL, BL)], sem).start()

        def wait_gathers(rowbuf, sem):
            pltpu.make_async_copy(emb_hbm.at[pl.ds(0, B)], rowbuf, sem).wait()

        def reduce_rows(rowbuf, accs):
            def body(i, accs):
                accs = list(accs)
                for j in range(KJ):
                    accs[j % 4] = accs[j % 4] + rowbuf[j * BL + i, :]
                return tuple(accs)
            return lax.fori_loop(0, BL, body, accs)

        idx_copy(0, idx0, si0).start()
        idx_copy(1, idx1, si1).start()
        idx_copy(0, idx0, si0).wait()
        start_gathers(idx0, rows0, sg0)

        zero = jnp.zeros((EMB,), jnp.float32)
        accs0 = (zero, zero, zero, zero)

        def round_pair(k, accs):
            wait_gathers(rows0, sg0)

            @pl.when(k + 1 < NK)
            def _():
                idx_copy(2 * k + 2, idx0, si0).start()

            idx_copy(2 * k + 1, idx1, si1).wait()
            start_gathers(idx1, rows1, sg1)
            accs = reduce_rows(rows0, accs)

            wait_gathers(rows1, sg1)

            @pl.when(k + 1 < NK)
            def _():
                idx_copy(2 * k + 3, idx1, si1).start()
                idx_copy(2 * k + 2, idx0, si0).wait()
                start_gathers(idx0, rows0, sg0)

            accs = reduce_rows(rows1, accs)
            return accs

        a0, a1, a2, a3 = lax.fori_loop(0, NK, round_pair, accs0)
        accv[...] = (a0 + a1) + (a2 + a3)
        pltpu.sync_copy(accv, out_hbm.at[wid])

    return sc_sum


def _tp_body(x_ref, o_ref):
    eye = jnp.eye(EMB, dtype=jnp.float32)
    o_ref[...] = lax.dot_general(
        x_ref[...], eye, (((0,), (0,)), ((), ())),
        preferred_element_type=jnp.float32)


def _build_table(embpT):
    return pl.pallas_call(
        _tp_body,
        grid=(VP // TC_C,),
        in_specs=[pl.BlockSpec((EMB, TC_C), lambda g: (0, g))],
        out_specs=pl.BlockSpec((TC_C, EMB), lambda g: (g, 0)),
        out_shape=jax.ShapeDtypeStruct((VP, EMB), jnp.float32),
    )(embpT)


def _tc_finish(partials, W, b2, inv_l):
    def body(p_ref, w_ref, b_ref, o_ref):
        pooled = jnp.sum(p_ref[...], axis=0, keepdims=True) * inv_l
        o_ref[...] = lax.dot_general(
            pooled, w_ref[...], (((1,), (1,)), ((), ())),
            preferred_element_type=jnp.float32) + b_ref[...]

    return pl.pallas_call(
        body,
        out_shape=jax.ShapeDtypeStruct((1, b2.shape[1]), jnp.float32),
    )(partials, W, b2)


def kernel(inputs, emb, W, b):
    L = inputs.shape[0]
    V = emb.shape[0]
    idx2d = inputs.astype(jnp.int32).reshape(L // BL, BL)
    embpT = jnp.pad(emb, ((0, VP - V), (0, 0))).T
    table = _build_table(embpT)
    partials = _make_sc_sum(L // BL)(idx2d, table)
    return _tc_finish(partials, W, b.reshape(1, -1), 1.0 / L)


# speedup vs baseline: 6.1056x; 2.3178x over previous
import functools

import jax
import jax.numpy as jnp
from jax import lax
from jax.experimental import pallas as pl
from jax.experimental.pallas import tpu as pltpu
from jax.experimental.pallas import tpu_sc as plsc

EMB = 16
NC = 2
NS = 16
NW = NC * NS
BL = 128
KJ = 16
B = KJ * BL
VP = 1 << 20
TC_C = 8192
TC_Q = TC_C // 8


def _tp_body(*refs):
    xs = refs[:8]
    o_ref = refs[8]
    xcat = jnp.concatenate([x[...] for x in xs], axis=0)
    eye = jnp.eye(128, dtype=jnp.float32)
    o_ref[...] = lax.dot_general(
        xcat, eye, (((0,), (0,)), ((), ())),
        preferred_element_type=jnp.float32)


def _build_table(embpT):
    def mk(j):
        return pl.BlockSpec((EMB, TC_Q), lambda g, j=j: (0, 8 * g + j))
    return pl.pallas_call(
        _tp_body,
        grid=(VP // TC_C,),
        in_specs=[mk(j) for j in range(8)],
        out_specs=pl.BlockSpec((TC_Q, 128), lambda g: (g, 0)),
        out_shape=jax.ShapeDtypeStruct((VP // 8, 128), jnp.float32),
        compiler_params=pltpu.CompilerParams(
            fuse_transposed_lhs_in_matmul=True),
    )(*([embpT] * 8))


@functools.lru_cache(maxsize=None)
def _make_sc_sum(n_rows: int):
    chunk_rows = n_rows // NW
    assert chunk_rows * NW == n_rows
    R = chunk_rows // KJ
    assert R * KJ == chunk_rows and R % 2 == 0
    NK = R // 2

    mesh = plsc.VectorSubcoreMesh(
        core_axis_name="c", subcore_axis_name="s",
        num_cores=NC, num_subcores=NS)

    @functools.partial(
        pl.kernel,
        out_type=jax.ShapeDtypeStruct((NW, EMB), jnp.float32),
        mesh=mesh,
        compiler_params=pltpu.CompilerParams(use_tc_tiling_on_sc=False),
        scratch_types=[
            pltpu.VMEM((KJ, BL), jnp.int32),
            pltpu.VMEM((KJ, BL), jnp.int32),
            pltpu.VMEM((KJ, BL), jnp.int32),
            pltpu.VMEM((KJ, BL), jnp.int32),
            pltpu.VMEM((B, EMB), jnp.float32),
            pltpu.VMEM((B, EMB), jnp.float32),
            pltpu.VMEM((EMB,), jnp.float32),
            pltpu.SemaphoreType.DMA,
            pltpu.SemaphoreType.DMA,
            pltpu.SemaphoreType.DMA,
            pltpu.SemaphoreType.DMA,
        ],
    )
    def sc_sum(idx_hbm, emb_hbm, out_hbm,
               idx0, idx1, rmp0, rmp1, rows0, rows1, accv,
               si0, si1, sg0, sg1):
        wid = lax.axis_index("s") * NC + lax.axis_index("c")
        base = wid * chunk_rows

        def idx_copy(r, buf, sem):
            return pltpu.make_async_copy(
                idx_hbm.at[pl.ds(base + r * KJ, KJ)], buf, sem)

        def remap(idxbuf, rmpbuf):
            def body(i, carry):
                jj = i // (BL // 16)
                off = (i % (BL // 16)) * 16
                v = idxbuf[jj, pl.ds(off, 16)]
                t = v & (TC_C - 1)
                bse = v - t
                j = lax.shift_right_logical(t, 10)
                q = t & (TC_Q - 1)
                rmpbuf[jj, pl.ds(off, 16)] = (
                    bse + lax.shift_left(q, 3) + j)
                return carry
            return lax.fori_loop(0, KJ * (BL // 16), body, 0)

        def start_gathers(rmpbuf, rowbuf, sem):
            for j in range(KJ):
                pltpu.make_async_copy(
                    emb_hbm.at[rmpbuf.at[j]],
                    rowbuf.at[pl.ds(j * BL, BL)], sem).start()

        def wait_gathers(rowbuf, sem):
            pltpu.make_async_copy(emb_hbm.at[pl.ds(0, B)], rowbuf, sem).wait()

        def reduce_rows(rowbuf, accs):
            def body(i, accs):
                accs = list(accs)
                for j in range(KJ):
                    accs[j % 4] = accs[j % 4] + rowbuf[j * BL + i, :]
                return tuple(accs)
            return lax.fori_loop(0, BL, body, accs)

        idx_copy(0, idx0, si0).start()
        idx_copy(1, idx1, si1).start()
        idx_copy(0, idx0, si0).wait()
        remap(idx0, rmp0)
        start_gathers(rmp0, rows0, sg0)

        zero = jnp.zeros((EMB,), jnp.float32)
        accs0 = (zero, zero, zero, zero)

        def round_pair(k, accs):
            idx_copy(2 * k + 1, idx1, si1).wait()
            remap(idx1, rmp1)
            wait_gathers(rows0, sg0)
            start_gathers(rmp1, rows1, sg1)

            @pl.when(k + 1 < NK)
            def _():
                idx_copy(2 * k + 2, idx0, si0).start()

            accs = reduce_rows(rows0, accs)
            wait_gathers(rows1, sg1)

            @pl.when(k + 1 < NK)
            def _():
                idx_copy(2 * k + 2, idx0, si0).wait()
                remap(idx0, rmp0)
                start_gathers(rmp0, rows0, sg0)
                idx_copy(2 * k + 3, idx1, si1).start()

            accs = reduce_rows(rows1, accs)
            return accs

        a0, a1, a2, a3 = lax.fori_loop(0, NK, round_pair, accs0)
        accv[...] = (a0 + a1) + (a2 + a3)
        pltpu.sync_copy(accv, out_hbm.at[wid])

    return sc_sum


def _tc_finish(partials, W, b2, inv_l):
    def body(p_ref, w_ref, b_ref, o_ref):
        pooled = jnp.sum(p_ref[...], axis=0, keepdims=True) * inv_l
        o_ref[...] = lax.dot_general(
            pooled, w_ref[...], (((1,), (1,)), ((), ())),
            preferred_element_type=jnp.float32) + b_ref[...]

    return pl.pallas_call(
        body,
        out_shape=jax.ShapeDtypeStruct((1, b2.shape[1]), jnp.float32),
    )(partials, W, b2)


def kernel(inputs, emb, W, b):
    L = inputs.shape[0]
    V = emb.shape[0]
    idx2d = inputs.astype(jnp.int32).reshape(L // BL, BL)
    embpT = jnp.pad(emb, ((0, VP - V), (0, 0))).T
    table = _build_table(embpT).reshape(VP, EMB)
    partials = _make_sc_sum(L // BL)(idx2d, table)
    return _tc_finish(partials, W, b.reshape(1, -1), 1.0 / L)


# speedup vs baseline: 7.0393x; 1.1529x over previous
import functools

import jax
import jax.numpy as jnp
from jax import lax
from jax.experimental import pallas as pl
from jax.experimental.pallas import tpu as pltpu
from jax.experimental.pallas import tpu_sc as plsc

EMB = 16
NC = 2
NS = 16
NW = NC * NS
BL = 128
KJ = 16
B = KJ * BL
VP = 1 << 20
TC_C = 32768
TC_Q = TC_C // 8


def _tp_body(*refs):
    xs = refs[:8]
    o_ref = refs[8]
    xcat = jnp.concatenate([x[...] for x in xs], axis=0)
    eye = jnp.eye(128, dtype=jnp.float32)
    o_ref[...] = lax.dot_general(
        xcat, eye, (((0,), (0,)), ((), ())),
        preferred_element_type=jnp.float32)


def _build_table(embpT):
    def mk(j):
        return pl.BlockSpec((EMB, TC_Q), lambda g, j=j: (0, 8 * g + j))
    return pl.pallas_call(
        _tp_body,
        grid=(VP // TC_C,),
        in_specs=[mk(j) for j in range(8)],
        out_specs=pl.BlockSpec((TC_Q, 128), lambda g: (g, 0)),
        out_shape=jax.ShapeDtypeStruct((VP // 8, 128), jnp.float32),
        compiler_params=pltpu.CompilerParams(
            fuse_transposed_lhs_in_matmul=True),
    )(*([embpT] * 8))


@functools.lru_cache(maxsize=None)
def _make_sc_sum(n_rows: int):
    chunk_rows = n_rows // NW
    assert chunk_rows * NW == n_rows
    R = chunk_rows // KJ
    assert R * KJ == chunk_rows and R % 2 == 0
    NK = R // 2

    mesh = plsc.VectorSubcoreMesh(
        core_axis_name="c", subcore_axis_name="s",
        num_cores=NC, num_subcores=NS)

    @functools.partial(
        pl.kernel,
        out_type=jax.ShapeDtypeStruct((NW, EMB), jnp.float32),
        mesh=mesh,
        compiler_params=pltpu.CompilerParams(use_tc_tiling_on_sc=False),
        scratch_types=[
            pltpu.VMEM((KJ, BL), jnp.int32),
            pltpu.VMEM((KJ, BL), jnp.int32),
            pltpu.VMEM((B, EMB), jnp.float32),
            pltpu.VMEM((B, EMB), jnp.float32),
            pltpu.VMEM((EMB,), jnp.float32),
            pltpu.SemaphoreType.DMA,
            pltpu.SemaphoreType.DMA,
            pltpu.SemaphoreType.DMA,
            pltpu.SemaphoreType.DMA,
        ],
    )
    def sc_sum(idx_hbm, emb_hbm, out_hbm,
               idx0, idx1, rows0, rows1, accv,
               si0, si1, sg0, sg1):
        wid = lax.axis_index("s") * NC + lax.axis_index("c")
        base = wid * chunk_rows

        def idx_copy(r, buf, sem):
            return pltpu.make_async_copy(
                idx_hbm.at[pl.ds(base + r * KJ, KJ)], buf, sem)

        def start_gathers(idxbuf, rowbuf, sem):
            for j in range(KJ):
                pltpu.make_async_copy(
                    emb_hbm.at[idxbuf.at[j]],
                    rowbuf.at[pl.ds(j * BL, BL)], sem).start()

        def wait_gathers(rowbuf, sem):
            pltpu.make_async_copy(emb_hbm.at[pl.ds(0, B)], rowbuf, sem).wait()

        def reduce_rows(rowbuf, accs):
            def body(i, accs):
                accs = list(accs)
                for j in range(KJ):
                    accs[j % 4] = accs[j % 4] + rowbuf[j * BL + i, :]
                    accs[j % 4] = (accs[j % 4]
                                   + rowbuf[j * BL + (BL // 2) + i, :])
                return tuple(accs)
            return lax.fori_loop(0, BL // 2, body, accs)

        idx_copy(0, idx0, si0).start()
        idx_copy(1, idx1, si1).start()
        idx_copy(0, idx0, si0).wait()
        start_gathers(idx0, rows0, sg0)

        zero = jnp.zeros((EMB,), jnp.float32)
        accs0 = (zero, zero, zero, zero)

        def round_pair(k, accs):
            idx_copy(2 * k + 1, idx1, si1).wait()
            wait_gathers(rows0, sg0)
            start_gathers(idx1, rows1, sg1)

            @pl.when(k + 1 < NK)
            def _():
                idx_copy(2 * k + 2, idx0, si0).start()

            accs = reduce_rows(rows0, accs)
            wait_gathers(rows1, sg1)

            @pl.when(k + 1 < NK)
            def _():
                idx_copy(2 * k + 2, idx0, si0).wait()
                start_gathers(idx0, rows0, sg0)
                idx_copy(2 * k + 3, idx1, si1).start()

            accs = reduce_rows(rows1, accs)
            return accs

        a0, a1, a2, a3 = lax.fori_loop(0, NK, round_pair, accs0)
        accv[...] = (a0 + a1) + (a2 + a3)
        pltpu.sync_copy(accv, out_hbm.at[wid])

    return sc_sum


def _tc_finish(partials, W, b2, inv_l):
    def body(p_ref, w_ref, b_ref, o_ref):
        pooled = jnp.sum(p_ref[...], axis=0, keepdims=True) * inv_l
        o_ref[...] = lax.dot_general(
            pooled, w_ref[...], (((1,), (1,)), ((), ())),
            preferred_element_type=jnp.float32) + b_ref[...]

    return pl.pallas_call(
        body,
        out_shape=jax.ShapeDtypeStruct((1, b2.shape[1]), jnp.float32),
    )(partials, W, b2)


def kernel(inputs, emb, W, b):
    L = inputs.shape[0]
    V = emb.shape[0]
    v = inputs.astype(jnp.int32)
    t = v & (TC_C - 1)
    q = t & (TC_Q - 1)
    j = lax.shift_right_logical(t, TC_Q.bit_length() - 1)
    rmp = (v - t) + lax.shift_left(q, 3) + j
    idx2d = rmp.reshape(L // BL, BL)
    embpT = jnp.pad(emb, ((0, VP - V), (0, 0))).T
    table = _build_table(embpT).reshape(VP, EMB)
    partials = _make_sc_sum(L // BL)(idx2d, table)
    return _tc_finish(partials, W, b.reshape(1, -1), 1.0 / L)


# speedup vs baseline: 7.1009x; 1.0087x over previous
import functools

import jax
import jax.numpy as jnp
from jax import lax
from jax.experimental import pallas as pl
from jax.experimental.pallas import tpu as pltpu
from jax.experimental.pallas import tpu_sc as plsc

EMB = 16
NC = 2
NS = 16
NW = NC * NS
BL = 128
KJ = 25
B = KJ * BL
VP = 1 << 20
TC_C = 32768
TC_Q = TC_C // 8


def _tp_body(*refs):
    xs = refs[:8]
    o_ref = refs[8]
    xcat = jnp.concatenate([x[...] for x in xs], axis=0)
    eye = jnp.eye(128, dtype=jnp.float32)
    o_ref[...] = lax.dot_general(
        xcat, eye, (((0,), (0,)), ((), ())),
        preferred_element_type=jnp.float32)


def _build_table(embpT):
    def mk(j):
        return pl.BlockSpec((EMB, TC_Q), lambda g, j=j: (0, 8 * g + j))
    return pl.pallas_call(
        _tp_body,
        grid=(VP // TC_C,),
        in_specs=[mk(j) for j in range(8)],
        out_specs=pl.BlockSpec((TC_Q, 128), lambda g: (g, 0)),
        out_shape=jax.ShapeDtypeStruct((VP // 8, 128), jnp.float32),
        compiler_params=pltpu.CompilerParams(
            fuse_transposed_lhs_in_matmul=True),
    )(*([embpT] * 8))


@functools.lru_cache(maxsize=None)
def _make_sc_sum(n_rows: int):
    chunk_rows = n_rows // NW
    assert chunk_rows * NW == n_rows
    R = chunk_rows // KJ
    assert R * KJ == chunk_rows and R % 2 == 0
    NK = R // 2

    mesh = plsc.VectorSubcoreMesh(
        core_axis_name="c", subcore_axis_name="s",
        num_cores=NC, num_subcores=NS)

    @functools.partial(
        pl.kernel,
        out_type=jax.ShapeDtypeStruct((NW, EMB), jnp.float32),
        mesh=mesh,
        compiler_params=pltpu.CompilerParams(use_tc_tiling_on_sc=False),
        scratch_types=[
            pltpu.VMEM((KJ, BL), jnp.int32),
            pltpu.VMEM((KJ, BL), jnp.int32),
            pltpu.VMEM((B, EMB), jnp.float32),
            pltpu.VMEM((B, EMB), jnp.float32),
            pltpu.VMEM((EMB,), jnp.float32),
            pltpu.SemaphoreType.DMA,
            pltpu.SemaphoreType.DMA,
            pltpu.SemaphoreType.DMA,
            pltpu.SemaphoreType.DMA,
        ],
    )
    def sc_sum(idx_hbm, emb_hbm, out_hbm,
               idx0, idx1, rows0, rows1, accv,
               si0, si1, sg0, sg1):
        wid = lax.axis_index("s") * NC + lax.axis_index("c")
        base = wid * chunk_rows

        def idx_copy(r, buf, sem):
            return pltpu.make_async_copy(
                idx_hbm.at[pl.ds(base + r * KJ, KJ)], buf, sem)

        def start_gathers(idxbuf, rowbuf, sem):
            for j in range(KJ):
                pltpu.make_async_copy(
                    emb_hbm.at[idxbuf.at[j]],
                    rowbuf.at[pl.ds(j * BL, BL)], sem).start()

        def wait_gathers(rowbuf, sem):
            pltpu.make_async_copy(emb_hbm.at[pl.ds(0, B)], rowbuf, sem).wait()

        def reduce_rows(rowbuf, accs):
            def body(i, accs):
                accs = list(accs)
                for j in range(KJ):
                    accs[j % 4] = accs[j % 4] + rowbuf[j * BL + i, :]
                    accs[j % 4] = (accs[j % 4]
                                   + rowbuf[j * BL + (BL // 2) + i, :])
                return tuple(accs)
            return lax.fori_loop(0, BL // 2, body, accs)

        idx_copy(0, idx0, si0).start()
        idx_copy(1, idx1, si1).start()
        idx_copy(0, idx0, si0).wait()
        start_gathers(idx0, rows0, sg0)

        zero = jnp.zeros((EMB,), jnp.float32)
        accs0 = (zero, zero, zero, zero)

        def round_pair(k, accs):
            idx_copy(2 * k + 1, idx1, si1).wait()
            wait_gathers(rows0, sg0)
            start_gathers(idx1, rows1, sg1)

            @pl.when(k + 1 < NK)
            def _():
                idx_copy(2 * k + 2, idx0, si0).start()

            accs = reduce_rows(rows0, accs)
            wait_gathers(rows1, sg1)

            @pl.when(k + 1 < NK)
            def _():
                idx_copy(2 * k + 2, idx0, si0).wait()
                start_gathers(idx0, rows0, sg0)
                idx_copy(2 * k + 3, idx1, si1).start()

            accs = reduce_rows(rows1, accs)
            return accs

        a0, a1, a2, a3 = lax.fori_loop(0, NK, round_pair, accs0)
        accv[...] = (a0 + a1) + (a2 + a3)
        pltpu.sync_copy(accv, out_hbm.at[wid])

    return sc_sum


def _tc_finish(partials, W, b2, inv_l):
    def body(p_ref, w_ref, b_ref, o_ref):
        pooled = jnp.sum(p_ref[...], axis=0, keepdims=True) * inv_l
        o_ref[...] = lax.dot_general(
            pooled, w_ref[...], (((1,), (1,)), ((), ())),
            preferred_element_type=jnp.float32) + b_ref[...]

    return pl.pallas_call(
        body,
        out_shape=jax.ShapeDtypeStruct((1, b2.shape[1]), jnp.float32),
    )(partials, W, b2)


def kernel(inputs, emb, W, b):
    L = inputs.shape[0]
    V = emb.shape[0]
    v = inputs.astype(jnp.int32)
    t = v & (TC_C - 1)
    q = t & (TC_Q - 1)
    j = lax.shift_right_logical(t, TC_Q.bit_length() - 1)
    rmp = (v - t) + lax.shift_left(q, 3) + j
    idx2d = rmp.reshape(L // BL, BL)
    embpT = jnp.pad(emb, ((0, VP - V), (0, 0))).T
    table = _build_table(embpT).reshape(VP, EMB)
    partials = _make_sc_sum(L // BL)(idx2d, table)
    return _tc_finish(partials, W, b.reshape(1, -1), 1.0 / L)


# speedup vs baseline: 7.7244x; 1.0878x over previous
import functools

import jax
import jax.numpy as jnp
from jax import lax
from jax.experimental import pallas as pl
from jax.experimental.pallas import tpu as pltpu
from jax.experimental.pallas import tpu_sc as plsc

EMB = 16
NC = 2
NS = 16
NW = NC * NS
BL = 128
KJ = 25
B = KJ * BL
VP = 1 << 20
TC_C = 32768
TC_Q = TC_C // 8


def _tp_body(*refs):
    xs = refs[:8]
    o_ref = refs[8]
    xcat = jnp.concatenate([x[...] for x in xs], axis=0)
    eye = jnp.eye(128, dtype=jnp.float32)
    o_ref[...] = lax.dot_general(
        xcat, eye, (((0,), (0,)), ((), ())),
        preferred_element_type=jnp.float32)


def _build_table(embpT):
    def mk(j):
        return pl.BlockSpec((EMB, TC_Q), lambda g, j=j: (0, 8 * g + j))
    return pl.pallas_call(
        _tp_body,
        grid=(VP // TC_C,),
        in_specs=[mk(j) for j in range(8)],
        out_specs=pl.BlockSpec((TC_Q, 128), lambda g: (g, 0)),
        out_shape=jax.ShapeDtypeStruct((VP // 8, 128), jnp.float32),
        compiler_params=pltpu.CompilerParams(
            fuse_transposed_lhs_in_matmul=True),
    )(*([embpT] * 8))


@functools.lru_cache(maxsize=None)
def _make_sc_sum(n_rows: int):
    chunk_rows = n_rows // NW
    assert chunk_rows * NW == n_rows
    R = chunk_rows // KJ
    assert R * KJ == chunk_rows and R % 2 == 0
    NK = R // 2

    mesh = plsc.VectorSubcoreMesh(
        core_axis_name="c", subcore_axis_name="s",
        num_cores=NC, num_subcores=NS)

    @functools.partial(
        pl.kernel,
        out_type=jax.ShapeDtypeStruct((NW, EMB), jnp.float32),
        mesh=mesh,
        compiler_params=pltpu.CompilerParams(use_tc_tiling_on_sc=False),
        scratch_types=[
            pltpu.VMEM((KJ, BL), jnp.int32),
            pltpu.VMEM((KJ, BL), jnp.int32),
            pltpu.VMEM((B, EMB), jnp.float32),
            pltpu.VMEM((B, EMB), jnp.float32),
            pltpu.VMEM((EMB,), jnp.float32),
            pltpu.SemaphoreType.DMA,
            pltpu.SemaphoreType.DMA,
            pltpu.SemaphoreType.DMA,
            pltpu.SemaphoreType.DMA,
        ],
    )
    def sc_sum(idx_hbm, emb_hbm, out_hbm,
               idx0, idx1, rows0, rows1, accv,
               si0, si1, sg0, sg1):
        wid = lax.axis_index("s") * NC + lax.axis_index("c")
        base = wid * chunk_rows

        def idx_copy(r, buf, sem):
            return pltpu.make_async_copy(
                idx_hbm.at[pl.ds(base + r * KJ, KJ)], buf, sem)

        def start_gathers(idxbuf, rowbuf, sem):
            for j in range(KJ):
                pltpu.make_async_copy(
                    emb_hbm.at[idxbuf.at[j]],
                    rowbuf.at[pl.ds(j * BL, BL)], sem).start()

        def wait_gathers(rowbuf, sem):
            pltpu.make_async_copy(emb_hbm.at[pl.ds(0, B)], rowbuf, sem).wait()

        def reduce_rows(rowbuf, accs):
            def body(i, accs):
                accs = list(accs)
                for j in range(KJ):
                    accs[j % 4] = accs[j % 4] + rowbuf[j * BL + i, :]
                    accs[j % 4] = (accs[j % 4]
                                   + rowbuf[j * BL + (BL // 2) + i, :])
                return tuple(accs)
            return lax.fori_loop(0, BL // 2, body, accs)

        idx_copy(0, idx0, si0).start()
        idx_copy(1, idx1, si1).start()
        idx_copy(0, idx0, si0).wait()
        start_gathers(idx0, rows0, sg0)

        zero = jnp.zeros((EMB,), jnp.float32)
        accs0 = (zero, zero, zero, zero)

        def round_pair(k, accs):
            idx_copy(2 * k + 1, idx1, si1).wait()
            start_gathers(idx1, rows1, sg1)
            wait_gathers(rows0, sg0)

            @pl.when(k + 1 < NK)
            def _():
                idx_copy(2 * k + 2, idx0, si0).start()

            accs = reduce_rows(rows0, accs)
            wait_gathers(rows1, sg1)

            @pl.when(k + 1 < NK)
            def _():
                idx_copy(2 * k + 2, idx0, si0).wait()
                start_gathers(idx0, rows0, sg0)
                idx_copy(2 * k + 3, idx1, si1).start()

            accs = reduce_rows(rows1, accs)
            return accs

        a0, a1, a2, a3 = lax.fori_loop(0, NK, round_pair, accs0)
        accv[...] = (a0 + a1) + (a2 + a3)
        pltpu.sync_copy(accv, out_hbm.at[wid])

    return sc_sum


def _tc_finish(partials, W, b2, inv_l):
    def body(p_ref, w_ref, b_ref, o_ref):
        pooled = jnp.sum(p_ref[...], axis=0, keepdims=True) * inv_l
        o_ref[...] = lax.dot_general(
            pooled, w_ref[...], (((1,), (1,)), ((), ())),
            preferred_element_type=jnp.float32) + b_ref[...]

    return pl.pallas_call(
        body,
        out_shape=jax.ShapeDtypeStruct((1, b2.shape[1]), jnp.float32),
    )(partials, W, b2)


def kernel(inputs, emb, W, b):
    L = inputs.shape[0]
    V = emb.shape[0]
    v = inputs.astype(jnp.int32)
    t = v & (TC_C - 1)
    q = t & (TC_Q - 1)
    j = lax.shift_right_logical(t, TC_Q.bit_length() - 1)
    rmp = (v - t) + lax.shift_left(q, 3) + j
    idx2d = rmp.reshape(L // BL, BL)
    embpT = jnp.pad(emb, ((0, VP - V), (0, 0))).T
    table = _build_table(embpT).reshape(VP, EMB)
    partials = _make_sc_sum(L // BL)(idx2d, table)
    return _tc_finish(partials, W, b.reshape(1, -1), 1.0 / L)


# speedup vs baseline: 7.9435x; 1.0284x over previous
import functools

import jax
import jax.numpy as jnp
from jax import lax
from jax.experimental import pallas as pl
from jax.experimental.pallas import tpu as pltpu
from jax.experimental.pallas import tpu_sc as plsc

EMB = 16
NC = 2
NS = 16
NW = NC * NS
BL = 128
KJ = 25
B = KJ * BL
VP = 1 << 20
TC_C = 32768
TC_Q = TC_C // 8


def _tp_body(*refs):
    xs = refs[:8]
    o_ref = refs[8]
    xcat = jnp.concatenate([x[...] for x in xs], axis=0)
    eye = jnp.eye(128, dtype=jnp.float32)
    o_ref[...] = lax.dot_general(
        xcat, eye, (((0,), (0,)), ((), ())),
        preferred_element_type=jnp.float32)


def _build_table(embpT):
    def mk(j):
        return pl.BlockSpec((EMB, TC_Q), lambda g, j=j: (0, 8 * g + j))
    return pl.pallas_call(
        _tp_body,
        grid=(VP // TC_C,),
        in_specs=[mk(j) for j in range(8)],
        out_specs=pl.BlockSpec((TC_Q, 128), lambda g: (g, 0)),
        out_shape=jax.ShapeDtypeStruct((VP // 8, 128), jnp.float32),
        compiler_params=pltpu.CompilerParams(
            fuse_transposed_lhs_in_matmul=True),
    )(*([embpT] * 8))


@functools.lru_cache(maxsize=None)
def _make_sc_sum(n_rows: int):
    chunk_rows = n_rows // NW
    assert chunk_rows * NW == n_rows
    R = chunk_rows // KJ
    assert R * KJ == chunk_rows and R % 2 == 0
    NK = R // 2

    mesh = plsc.VectorSubcoreMesh(
        core_axis_name="c", subcore_axis_name="s",
        num_cores=NC, num_subcores=NS)

    @functools.partial(
        pl.kernel,
        out_type=jax.ShapeDtypeStruct((NW, EMB), jnp.float32),
        mesh=mesh,
        compiler_params=pltpu.CompilerParams(use_tc_tiling_on_sc=False),
        scratch_types=[
            pltpu.VMEM((KJ, BL), jnp.int32),
            pltpu.VMEM((KJ, BL), jnp.int32),
            pltpu.VMEM((B, EMB), jnp.float32),
            pltpu.VMEM((B, EMB), jnp.float32),
            pltpu.VMEM((EMB,), jnp.float32),
            pltpu.SemaphoreType.DMA,
            pltpu.SemaphoreType.DMA,
            pltpu.SemaphoreType.DMA,
            pltpu.SemaphoreType.DMA,
        ],
    )
    def sc_sum(idx_hbm, emb_hbm, out_hbm,
               idx0, idx1, rows0, rows1, accv,
               si0, si1, sg0, sg1):
        wid = lax.axis_index("s") * NC + lax.axis_index("c")
        base = wid * chunk_rows

        def idx_copy(r, buf, sem):
            return pltpu.make_async_copy(
                idx_hbm.at[pl.ds(base + r * KJ, KJ)], buf, sem)

        def start_gathers(idxbuf, rowbuf, sem):
            for j in range(KJ):
                pltpu.make_async_copy(
                    emb_hbm.at[idxbuf.at[j]],
                    rowbuf.at[pl.ds(j * BL, BL)], sem).start()

        def wait_gathers(rowbuf, sem):
            pltpu.make_async_copy(emb_hbm.at[pl.ds(0, B)], rowbuf, sem).wait()

        def reduce_rows(rowbuf, accs):
            def body(i, accs):
                accs = list(accs)
                for j in range(KJ):
                    accs[j % 4] = accs[j % 4] + rowbuf[j * BL + i, :]
                    accs[j % 4] = (accs[j % 4]
                                   + rowbuf[j * BL + (BL // 2) + i, :])
                return tuple(accs)
            return lax.fori_loop(0, BL // 2, body, accs)

        idx_copy(0, idx0, si0).start()
        idx_copy(1, idx1, si1).start()
        idx_copy(0, idx0, si0).wait()
        start_gathers(idx0, rows0, sg0)

        zero = jnp.zeros((EMB,), jnp.float32)
        accs0 = (zero, zero, zero, zero)

        def round_pair(k, accs):
            idx_copy(2 * k + 1, idx1, si1).wait()
            start_gathers(idx1, rows1, sg1)
            wait_gathers(rows0, sg0)

            @pl.when(k + 1 < NK)
            def _():
                idx_copy(2 * k + 2, idx0, si0).start()

            accs = reduce_rows(rows0, accs)

            @pl.when(k + 1 < NK)
            def _():
                idx_copy(2 * k + 2, idx0, si0).wait()
                start_gathers(idx0, rows0, sg0)
                idx_copy(2 * k + 3, idx1, si1).start()

            wait_gathers(rows1, sg1)
            accs = reduce_rows(rows1, accs)
            return accs

        a0, a1, a2, a3 = lax.fori_loop(0, NK, round_pair, accs0)
        accv[...] = (a0 + a1) + (a2 + a3)
        pltpu.sync_copy(accv, out_hbm.at[wid])

    return sc_sum


def _tc_finish(partials, W, b2, inv_l):
    def body(p_ref, w_ref, b_ref, o_ref):
        pooled = jnp.sum(p_ref[...], axis=0, keepdims=True) * inv_l
        o_ref[...] = lax.dot_general(
            pooled, w_ref[...], (((1,), (1,)), ((), ())),
            preferred_element_type=jnp.float32) + b_ref[...]

    return pl.pallas_call(
        body,
        out_shape=jax.ShapeDtypeStruct((1, b2.shape[1]), jnp.float32),
    )(partials, W, b2)


def kernel(inputs, emb, W, b):
    L = inputs.shape[0]
    V = emb.shape[0]
    v = inputs.astype(jnp.int32)
    t = v & (TC_C - 1)
    q = t & (TC_Q - 1)
    j = lax.shift_right_logical(t, TC_Q.bit_length() - 1)
    rmp = (v - t) + lax.shift_left(q, 3) + j
    idx2d = rmp.reshape(L // BL, BL)
    embpT = jnp.pad(emb, ((0, VP - V), (0, 0))).T
    table = _build_table(embpT).reshape(VP, EMB)
    partials = _make_sc_sum(L // BL)(idx2d, table)
    return _tc_finish(partials, W, b.reshape(1, -1), 1.0 / L)
